# baseline (speedup 1.0000x reference)
import functools

import jax
import jax.numpy as jnp
from jax import lax
from jax.experimental import pallas as pl
from jax.experimental.pallas import tpu as pltpu
from jax.experimental.pallas import tpu_sc as plsc


def _proj_body(nf_ref, wst_ref, wdt_ref, ps_ref, pd_ref):
    x = nf_ref[...]
    ps_ref[...] = jnp.dot(x, wst_ref[...], preferred_element_type=jnp.float32)
    pd_ref[...] = jnp.dot(x, wdt_ref[...], preferred_element_type=jnp.float32)


def _project_nodes(nfeat, wst, wdt, block_n):
    n, d = nfeat.shape
    h = wst.shape[1]
    grid = n // block_n
    return pl.pallas_call(
        _proj_body,
        grid=(grid,),
        in_specs=[
            pl.BlockSpec((block_n, d), lambda i: (i, 0)),
            pl.BlockSpec((d, h), lambda i: (0, 0)),
            pl.BlockSpec((d, h), lambda i: (0, 0)),
        ],
        out_specs=[
            pl.BlockSpec((block_n, h), lambda i: (i, 0)),
            pl.BlockSpec((block_n, h), lambda i: (i, 0)),
        ],
        out_shape=[
            jax.ShapeDtypeStruct((n, h), jnp.float32),
            jax.ShapeDtypeStruct((n, h), jnp.float32),
        ],
    )(nfeat, wst, wdt)


def _make_sc_gather(e_total, h, n_workers, chunk, base_off=0, nslots=4, lead=2):
    per_w = e_total // n_workers
    n_full = per_w // chunk
    tail = per_w - n_full * chunk
    n_outer = n_full // nslots
    assert n_full % nslots == 0 and chunk % 8 == 0 and chunk <= 128
    assert tail % 8 == 0 and tail <= 128
    mesh = plsc.VectorSubcoreMesh(core_axis_name="c", subcore_axis_name="s")

    @functools.partial(
        pl.kernel,
        mesh=mesh,
        out_type=jax.ShapeDtypeStruct((e_total, h), jnp.float32),
        scratch_types=[
            pltpu.VMEM((per_w,), jnp.int32),
            pltpu.VMEM((per_w,), jnp.int32),
            pltpu.VMEM((nslots, chunk, h), jnp.float32),
            pltpu.VMEM((nslots, chunk, h), jnp.float32),
            pltpu.SemaphoreType.DMA((nslots,)),
            pltpu.SemaphoreType.DMA((nslots,)),
            pltpu.SemaphoreType.DMA((nslots,)),
        ],
    )
    def gather_k(ps_hbm, pd_hbm, src_hbm, dst_hbm, g_hbm,
                 idx_s, idx_d, rows_s, rows_d, sem_gs, sem_gd, sem_w):
        n_cores = lax.axis_size("c")
        wid = lax.axis_index("s") * n_cores + lax.axis_index("c")
        base = wid * per_w

        pltpu.sync_copy(src_hbm.at[pl.ds(base_off + base, per_w)], idx_s)
        pltpu.sync_copy(dst_hbm.at[pl.ds(base_off + base, per_w)], idx_d)

        def fire_gather(t, slot):
            pltpu.async_copy(ps_hbm.at[idx_s.at[pl.ds(t * chunk, chunk)]],
                             rows_s.at[slot], sem_gs.at[slot])
            pltpu.async_copy(pd_hbm.at[idx_d.at[pl.ds(t * chunk, chunk)]],
                             rows_d.at[slot], sem_gd.at[slot])

        def wait_gather(t, slot):
            pltpu.make_async_copy(
                ps_hbm.at[idx_s.at[pl.ds(t * chunk, chunk)]],
                rows_s.at[slot], sem_gs.at[slot]).wait()
            pltpu.make_async_copy(
                pd_hbm.at[idx_d.at[pl.ds(t * chunk, chunk)]],
                rows_d.at[slot], sem_gd.at[slot]).wait()

        def fire_write(t, slot):
            pltpu.async_copy(rows_s.at[slot],
                             g_hbm.at[pl.ds(base + t * chunk, chunk)],
                             sem_w.at[slot])

        def wait_write(t_old, slot):
            pltpu.make_async_copy(
                rows_s.at[slot],
                g_hbm.at[pl.ds(base + t_old * chunk, chunk)],
                sem_w.at[slot]).wait()

        def add_block(slot, nrows):
            def add_row(j, c2):
                for k in range(h // 16):
                    sl = pl.ds(k * 16, 16)
                    rows_s[slot, j, sl] = rows_s[slot, j, sl] + rows_d[slot, j, sl]
                return c2
            lax.fori_loop(0, nrows, add_row, 0, unroll=2)

        for t in range(lead):
            fire_gather(t, t)

        def outer(j, carry):
            for b in range(nslots):
                i = j * nslots + b
                wait_gather(i, b)
                add_block(b, chunk)
                fire_write(i, b)
                sb = (b + lead) % nslots
                t = i + lead
                if b + lead < nslots:
                    @pl.when(j > 0)
                    def _():
                        wait_write(t - nslots, sb)
                    fire_gather(t, sb)
                else:
                    @pl.when(j < n_outer - 1)
                    def _():
                        wait_write(t - nslots, sb)
                        fire_gather(t, sb)
            return carry

        lax.fori_loop(0, n_outer, outer, 0)

        for b in range(nslots):
            wait_write(n_full - nslots + b, b)

        if tail:
            toff = n_full * chunk
            cp_s = pltpu.async_copy(
                ps_hbm.at[idx_s.at[pl.ds(toff, tail)]],
                rows_s.at[0, pl.ds(0, tail)], sem_gs.at[0])
            cp_d = pltpu.async_copy(
                pd_hbm.at[idx_d.at[pl.ds(toff, tail)]],
                rows_d.at[0, pl.ds(0, tail)], sem_gd.at[0])
            cp_s.wait()
            cp_d.wait()
            add_block(0, tail)
            pltpu.sync_copy(rows_s.at[0, pl.ds(0, tail)],
                            g_hbm.at[pl.ds(base + toff, tail)])

    return gather_k


def _dense_body(e_ref, g_ref, wet_ref, wot_ref, be_ref, bo_ref, lg_ref,
                lb_ref, out_ref):
    e = e_ref[...]
    pre = jnp.dot(e, wet_ref[...], preferred_element_type=jnp.float32)
    pre = pre + g_ref[...] + be_ref[...]
    act = pre * jax.nn.sigmoid(pre)
    o = jnp.dot(act, wot_ref[...], preferred_element_type=jnp.float32)
    o = o + bo_ref[...]
    mu = jnp.mean(o, axis=-1, keepdims=True)
    c = o - mu
    var = jnp.mean(c * c, axis=-1, keepdims=True)
    out_ref[...] = c * lax.rsqrt(var + 1e-5) * lg_ref[...] + lb_ref[...] + e


def _dense_body_aliased(e_ref, g_ref, wet_ref, wot_ref, be_ref, bo_ref,
                        lg_ref, lb_ref, prev_ref, out_ref):
    del prev_ref
    _dense_body(e_ref, g_ref, wet_ref, wot_ref, be_ref, bo_ref, lg_ref,
                lb_ref, out_ref)


def _dense_mlp_stripe(efeat, g_stripe, wet, wot, be, bo, lg, lb, prev,
                      stripe_base, block_e):
    e_total, d = efeat.shape
    h = wet.shape[1]
    stripe = g_stripe.shape[0]
    grid = stripe // block_e
    b0 = stripe_base // block_e
    full = lambda i: (0, 0)
    in_specs = [
        pl.BlockSpec((block_e, d), lambda i: (b0 + i, 0)),
        pl.BlockSpec((block_e, h), lambda i: (i, 0)),
        pl.BlockSpec((d, h), full),
        pl.BlockSpec((h, d), full),
        pl.BlockSpec((1, h), full),
        pl.BlockSpec((1, d), full),
        pl.BlockSpec((1, d), full),
        pl.BlockSpec((1, d), full),
    ]
    args = [efeat, g_stripe, wet, wot, be, bo, lg, lb]
    kwargs = {}
    body = _dense_body
    if prev is not None:
        in_specs.append(pl.BlockSpec(memory_space=pl.ANY))
        args.append(prev)
        kwargs["input_output_aliases"] = {8: 0}
        body = _dense_body_aliased
    return pl.pallas_call(
        body,
        grid=(grid,),
        in_specs=in_specs,
        out_specs=pl.BlockSpec((block_e, d), lambda i: (b0 + i, 0)),
        out_shape=jax.ShapeDtypeStruct((e_total, d), jnp.float32),
        **kwargs,
    )(*args)


def kernel(efeat, nfeat, edge_index, W_e, b_e, W_s, W_d, W_o, b_o, ln_g, ln_b):
    e_total, d = efeat.shape
    n = nfeat.shape[0]
    h = W_e.shape[0]

    src = edge_index[0]
    dst = edge_index[1]

    ps, pd = _project_nodes(nfeat, W_s.T, W_d.T, block_n=1000)

    n_workers = 32
    n_stripes = 5
    stripe = e_total // n_stripes

    gs = []
    for i in range(n_stripes):
        gather = _make_sc_gather(stripe, h, n_workers, chunk=128,
                                 base_off=i * stripe, nslots=3, lead=2)
        gs.append(gather(ps, pd, src, dst))

    wet, wot = W_e.T, W_o.T
    be, bo = b_e.reshape(1, h), b_o.reshape(1, d)
    lg, lb = ln_g.reshape(1, d), ln_b.reshape(1, d)
    out = None
    for i in range(n_stripes):
        out = _dense_mlp_stripe(efeat, gs[i], wet, wot, be, bo, lg, lb,
                                out, i * stripe, block_e=8000)
    return (out, nfeat)


# speedup vs baseline: 1.0784x; 1.0784x over previous
import functools

import jax
import jax.numpy as jnp
from jax import lax
from jax.experimental import pallas as pl
from jax.experimental.pallas import tpu as pltpu
from jax.experimental.pallas import tpu_sc as plsc


def _proj_body(nf_ref, wst_ref, wdt_ref, ps_ref, pd_ref):
    x = nf_ref[...]
    ps_ref[...] = jnp.dot(x, wst_ref[...], preferred_element_type=jnp.float32)
    pd_ref[...] = jnp.dot(x, wdt_ref[...], preferred_element_type=jnp.float32)


def _project_nodes(nfeat, wst, wdt, block_n):
    n, d = nfeat.shape
    h = wst.shape[1]
    grid = n // block_n
    return pl.pallas_call(
        _proj_body,
        grid=(grid,),
        in_specs=[
            pl.BlockSpec((block_n, d), lambda i: (i, 0)),
            pl.BlockSpec((d, h), lambda i: (0, 0)),
            pl.BlockSpec((d, h), lambda i: (0, 0)),
        ],
        out_specs=[
            pl.BlockSpec((block_n, h), lambda i: (i, 0)),
            pl.BlockSpec((block_n, h), lambda i: (i, 0)),
        ],
        out_shape=[
            jax.ShapeDtypeStruct((n, h), jnp.float32),
            jax.ShapeDtypeStruct((n, h), jnp.float32),
        ],
    )(nfeat, wst, wdt)


def _make_sc_gather(e_total, h, n_workers, chunk, base_off=0, nslots=4, lead=2):
    per_w = e_total // n_workers
    n_full = per_w // chunk
    tail = per_w - n_full * chunk
    n_outer = n_full // nslots
    assert n_full % nslots == 0 and chunk % 8 == 0 and chunk <= 128
    assert tail % 8 == 0 and tail <= 128
    mesh = plsc.VectorSubcoreMesh(core_axis_name="c", subcore_axis_name="s")

    @functools.partial(
        pl.kernel,
        mesh=mesh,
        out_type=jax.ShapeDtypeStruct((e_total, h), jnp.float32),
        scratch_types=[
            pltpu.VMEM((per_w,), jnp.int32),
            pltpu.VMEM((per_w,), jnp.int32),
            pltpu.VMEM((nslots, chunk, h), jnp.float32),
            pltpu.VMEM((nslots, chunk, h), jnp.float32),
            pltpu.SemaphoreType.DMA((nslots,)),
            pltpu.SemaphoreType.DMA((nslots,)),
            pltpu.SemaphoreType.DMA((nslots,)),
        ],
    )
    def gather_k(ps_hbm, pd_hbm, src_hbm, dst_hbm, g_hbm,
                 idx_s, idx_d, rows_s, rows_d, sem_gs, sem_gd, sem_w):
        n_cores = lax.axis_size("c")
        wid = lax.axis_index("s") * n_cores + lax.axis_index("c")
        base = wid * per_w

        cp_is = pltpu.async_copy(src_hbm.at[pl.ds(base_off + base, per_w)],
                                 idx_s, sem_gs.at[0])
        cp_id = pltpu.async_copy(dst_hbm.at[pl.ds(base_off + base, per_w)],
                                 idx_d, sem_gd.at[0])
        cp_is.wait()
        cp_id.wait()

        def fire_gather(t, slot):
            pltpu.async_copy(ps_hbm.at[idx_s.at[pl.ds(t * chunk, chunk)]],
                             rows_s.at[slot], sem_gs.at[slot])
            pltpu.async_copy(pd_hbm.at[idx_d.at[pl.ds(t * chunk, chunk)]],
                             rows_d.at[slot], sem_gd.at[slot])

        def wait_gather(t, slot):
            pltpu.make_async_copy(
                ps_hbm.at[idx_s.at[pl.ds(t * chunk, chunk)]],
                rows_s.at[slot], sem_gs.at[slot]).wait()
            pltpu.make_async_copy(
                pd_hbm.at[idx_d.at[pl.ds(t * chunk, chunk)]],
                rows_d.at[slot], sem_gd.at[slot]).wait()

        def fire_write(t, slot):
            pltpu.async_copy(rows_s.at[slot],
                             g_hbm.at[pl.ds(base + t * chunk, chunk)],
                             sem_w.at[slot])

        def wait_write(t_old, slot):
            pltpu.make_async_copy(
                rows_s.at[slot],
                g_hbm.at[pl.ds(base + t_old * chunk, chunk)],
                sem_w.at[slot]).wait()

        def add_block(slot, nrows):
            def add_row(j, c2):
                for k in range(h // 16):
                    sl = pl.ds(k * 16, 16)
                    rows_s[slot, j, sl] = rows_s[slot, j, sl] + rows_d[slot, j, sl]
                return c2
            lax.fori_loop(0, nrows, add_row, 0, unroll=2)

        for t in range(lead):
            fire_gather(t, t)

        def outer(j, carry):
            for b in range(nslots):
                i = j * nslots + b
                wait_gather(i, b)
                add_block(b, chunk)
                fire_write(i, b)
                sb = (b + lead) % nslots
                t = i + lead
                if b + lead < nslots:
                    @pl.when(j > 0)
                    def _():
                        wait_write(t - nslots, sb)
                    fire_gather(t, sb)
                else:
                    @pl.when(j < n_outer - 1)
                    def _():
                        wait_write(t - nslots, sb)
                        fire_gather(t, sb)
            return carry

        lax.fori_loop(0, n_outer, outer, 0)

        for b in range(nslots):
            wait_write(n_full - nslots + b, b)

        if tail:
            toff = n_full * chunk
            cp_s = pltpu.async_copy(
                ps_hbm.at[idx_s.at[pl.ds(toff, tail)]],
                rows_s.at[0, pl.ds(0, tail)], sem_gs.at[0])
            cp_d = pltpu.async_copy(
                pd_hbm.at[idx_d.at[pl.ds(toff, tail)]],
                rows_d.at[0, pl.ds(0, tail)], sem_gd.at[0])
            cp_s.wait()
            cp_d.wait()
            add_block(0, tail)
            pltpu.sync_copy(rows_s.at[0, pl.ds(0, tail)],
                            g_hbm.at[pl.ds(base + toff, tail)])

    return gather_k


def _dense_body(e_ref, g_ref, wet_ref, wot_ref, be_ref, bo_ref, lg_ref,
                lb_ref, out_ref):
    e = e_ref[...]
    pre = jnp.dot(e, wet_ref[...], preferred_element_type=jnp.float32)
    pre = pre + g_ref[...] + be_ref[...]
    act = pre * jax.nn.sigmoid(pre)
    o = jnp.dot(act, wot_ref[...], preferred_element_type=jnp.float32)
    o = o + bo_ref[...]
    mu = jnp.mean(o, axis=-1, keepdims=True)
    c = o - mu
    var = jnp.mean(c * c, axis=-1, keepdims=True)
    out_ref[...] = c * lax.rsqrt(var + 1e-5) * lg_ref[...] + lb_ref[...] + e


def _dense_body_aliased(e_ref, g_ref, wet_ref, wot_ref, be_ref, bo_ref,
                        lg_ref, lb_ref, prev_ref, out_ref):
    del prev_ref
    _dense_body(e_ref, g_ref, wet_ref, wot_ref, be_ref, bo_ref, lg_ref,
                lb_ref, out_ref)


def _dense_mlp_stripe(efeat, g_stripe, wet, wot, be, bo, lg, lb, prev,
                      stripe_base, block_e):
    e_total, d = efeat.shape
    h = wet.shape[1]
    stripe = g_stripe.shape[0]
    grid = stripe // block_e
    b0 = stripe_base // block_e
    full = lambda i: (0, 0)
    in_specs = [
        pl.BlockSpec((block_e, d), lambda i: (b0 + i, 0)),
        pl.BlockSpec((block_e, h), lambda i: (i, 0)),
        pl.BlockSpec((d, h), full),
        pl.BlockSpec((h, d), full),
        pl.BlockSpec((1, h), full),
        pl.BlockSpec((1, d), full),
        pl.BlockSpec((1, d), full),
        pl.BlockSpec((1, d), full),
    ]
    args = [efeat, g_stripe, wet, wot, be, bo, lg, lb]
    kwargs = {}
    body = _dense_body
    if prev is not None:
        in_specs.append(pl.BlockSpec(memory_space=pl.ANY))
        args.append(prev)
        kwargs["input_output_aliases"] = {8: 0}
        body = _dense_body_aliased
    return pl.pallas_call(
        body,
        grid=(grid,),
        in_specs=in_specs,
        out_specs=pl.BlockSpec((block_e, d), lambda i: (b0 + i, 0)),
        out_shape=jax.ShapeDtypeStruct((e_total, d), jnp.float32),
        **kwargs,
    )(*args)


def kernel(efeat, nfeat, edge_index, W_e, b_e, W_s, W_d, W_o, b_o, ln_g, ln_b):
    e_total, d = efeat.shape
    n = nfeat.shape[0]
    h = W_e.shape[0]

    src = edge_index[0]
    dst = edge_index[1]

    ps, pd = _project_nodes(nfeat, W_s.T, W_d.T, block_n=1000)

    n_workers = 32
    n_stripes = 5
    stripe = e_total // n_stripes

    gs = []
    for i in range(n_stripes):
        gather = _make_sc_gather(stripe, h, n_workers, chunk=80,
                                 base_off=i * stripe, nslots=5, lead=4)
        gs.append(gather(ps, pd, src, dst))

    wet, wot = W_e.T, W_o.T
    be, bo = b_e.reshape(1, h), b_o.reshape(1, d)
    lg, lb = ln_g.reshape(1, d), ln_b.reshape(1, d)
    out = None
    for i in range(n_stripes):
        out = _dense_mlp_stripe(efeat, gs[i], wet, wot, be, bo, lg, lb,
                                out, i * stripe, block_e=8000)
    return (out, nfeat)


# speedup vs baseline: 1.1181x; 1.0368x over previous
import functools

import jax
import jax.numpy as jnp
from jax import lax
from jax.experimental import pallas as pl
from jax.experimental.pallas import tpu as pltpu
from jax.experimental.pallas import tpu_sc as plsc


def _proj_body(nf_ref, wst_ref, wdt_ref, ps_ref, pd_ref):
    x = nf_ref[...]
    ps_ref[...] = jnp.dot(x, wst_ref[...], preferred_element_type=jnp.float32)
    pd_ref[...] = jnp.dot(x, wdt_ref[...], preferred_element_type=jnp.float32)


def _project_nodes(nfeat, wst, wdt, block_n):
    n, d = nfeat.shape
    h = wst.shape[1]
    grid = n // block_n
    return pl.pallas_call(
        _proj_body,
        grid=(grid,),
        in_specs=[
            pl.BlockSpec((block_n, d), lambda i: (i, 0)),
            pl.BlockSpec((d, h), lambda i: (0, 0)),
            pl.BlockSpec((d, h), lambda i: (0, 0)),
        ],
        out_specs=[
            pl.BlockSpec((block_n, h), lambda i: (i, 0)),
            pl.BlockSpec((block_n, h), lambda i: (i, 0)),
        ],
        out_shape=[
            jax.ShapeDtypeStruct((n, h), jnp.float32),
            jax.ShapeDtypeStruct((n, h), jnp.float32),
        ],
    )(nfeat, wst, wdt)


def _make_sc_gather(e_total, h, n_workers, chunk, base_off=0, nslots=4, lead=2):
    per_w = e_total // n_workers
    n_full = per_w // chunk
    tail = per_w - n_full * chunk
    n_outer = n_full // nslots
    assert n_full % nslots == 0 and chunk % 8 == 0 and chunk <= 128
    assert tail % 8 == 0 and tail <= 128
    mesh = plsc.VectorSubcoreMesh(core_axis_name="c", subcore_axis_name="s")

    @functools.partial(
        pl.kernel,
        mesh=mesh,
        out_type=jax.ShapeDtypeStruct((e_total, h), jnp.float32),
        scratch_types=[
            pltpu.VMEM((per_w,), jnp.int32),
            pltpu.VMEM((per_w,), jnp.int32),
            pltpu.VMEM((nslots, chunk, h), jnp.float32),
            pltpu.VMEM((nslots, chunk, h), jnp.float32),
            pltpu.SemaphoreType.DMA((nslots,)),
            pltpu.SemaphoreType.DMA((nslots,)),
            pltpu.SemaphoreType.DMA((nslots,)),
        ],
    )
    def gather_k(ps_hbm, pd_hbm, src_hbm, dst_hbm, g_hbm,
                 idx_s, idx_d, rows_s, rows_d, sem_gs, sem_gd, sem_w):
        n_cores = lax.axis_size("c")
        wid = lax.axis_index("s") * n_cores + lax.axis_index("c")
        base = wid * per_w

        cp_is = pltpu.async_copy(src_hbm.at[pl.ds(base_off + base, per_w)],
                                 idx_s, sem_gs.at[0])
        cp_id = pltpu.async_copy(dst_hbm.at[pl.ds(base_off + base, per_w)],
                                 idx_d, sem_gd.at[0])
        cp_is.wait()
        cp_id.wait()

        def fire_gather(t, slot):
            pltpu.async_copy(ps_hbm.at[idx_s.at[pl.ds(t * chunk, chunk)]],
                             rows_s.at[slot], sem_gs.at[slot])
            pltpu.async_copy(pd_hbm.at[idx_d.at[pl.ds(t * chunk, chunk)]],
                             rows_d.at[slot], sem_gd.at[slot])

        def wait_gather(t, slot):
            pltpu.make_async_copy(
                ps_hbm.at[idx_s.at[pl.ds(t * chunk, chunk)]],
                rows_s.at[slot], sem_gs.at[slot]).wait()
            pltpu.make_async_copy(
                pd_hbm.at[idx_d.at[pl.ds(t * chunk, chunk)]],
                rows_d.at[slot], sem_gd.at[slot]).wait()

        def fire_write(t, slot):
            pltpu.async_copy(rows_s.at[slot],
                             g_hbm.at[pl.ds(base + t * chunk, chunk)],
                             sem_w.at[slot])

        def wait_write(t_old, slot):
            pltpu.make_async_copy(
                rows_s.at[slot],
                g_hbm.at[pl.ds(base + t_old * chunk, chunk)],
                sem_w.at[slot]).wait()

        def add_block(slot, nrows):
            def add_row(j, c2):
                for k in range(h // 16):
                    sl = pl.ds(k * 16, 16)
                    rows_s[slot, j, sl] = rows_s[slot, j, sl] + rows_d[slot, j, sl]
                return c2
            lax.fori_loop(0, nrows, add_row, 0, unroll=2)

        for t in range(lead):
            fire_gather(t, t)

        def outer(j, carry):
            for b in range(nslots):
                i = j * nslots + b
                wait_gather(i, b)
                add_block(b, chunk)
                fire_write(i, b)
                sb = (b + lead) % nslots
                t = i + lead
                if b + lead < nslots:
                    @pl.when(j > 0)
                    def _():
                        wait_write(t - nslots, sb)
                    fire_gather(t, sb)
                else:
                    @pl.when(j < n_outer - 1)
                    def _():
                        wait_write(t - nslots, sb)
                        fire_gather(t, sb)
            return carry

        lax.fori_loop(0, n_outer, outer, 0)

        for b in range(nslots):
            wait_write(n_full - nslots + b, b)

        if tail:
            toff = n_full * chunk
            cp_s = pltpu.async_copy(
                ps_hbm.at[idx_s.at[pl.ds(toff, tail)]],
                rows_s.at[0, pl.ds(0, tail)], sem_gs.at[0])
            cp_d = pltpu.async_copy(
                pd_hbm.at[idx_d.at[pl.ds(toff, tail)]],
                rows_d.at[0, pl.ds(0, tail)], sem_gd.at[0])
            cp_s.wait()
            cp_d.wait()
            add_block(0, tail)
            pltpu.sync_copy(rows_s.at[0, pl.ds(0, tail)],
                            g_hbm.at[pl.ds(base + toff, tail)])

    return gather_k


def _dense_body(e_ref, g_ref, wet_ref, wot_ref, be_ref, bo_ref, lg_ref,
                lb_ref, out_ref):
    e = e_ref[...]
    pre = jnp.dot(e, wet_ref[...], preferred_element_type=jnp.float32)
    pre = pre + g_ref[...] + be_ref[...]
    act = pre * jax.nn.sigmoid(pre)
    o = jnp.dot(act, wot_ref[...], preferred_element_type=jnp.float32)
    o = o + bo_ref[...]
    mu = jnp.mean(o, axis=-1, keepdims=True)
    c = o - mu
    var = jnp.mean(c * c, axis=-1, keepdims=True)
    out_ref[...] = c * lax.rsqrt(var + 1e-5) * lg_ref[...] + lb_ref[...] + e


def _dense_body_aliased(e_ref, g_ref, wet_ref, wot_ref, be_ref, bo_ref,
                        lg_ref, lb_ref, prev_ref, out_ref):
    del prev_ref
    _dense_body(e_ref, g_ref, wet_ref, wot_ref, be_ref, bo_ref, lg_ref,
                lb_ref, out_ref)


def _dense_mlp_stripe(efeat, g_stripe, wet, wot, be, bo, lg, lb, prev,
                      stripe_base, block_e):
    e_total, d = efeat.shape
    h = wet.shape[1]
    stripe = g_stripe.shape[0]
    grid = stripe // block_e
    b0 = stripe_base // block_e
    full = lambda i: (0, 0)
    in_specs = [
        pl.BlockSpec((block_e, d), lambda i: (b0 + i, 0)),
        pl.BlockSpec((block_e, h), lambda i: (i, 0)),
        pl.BlockSpec((d, h), full),
        pl.BlockSpec((h, d), full),
        pl.BlockSpec((1, h), full),
        pl.BlockSpec((1, d), full),
        pl.BlockSpec((1, d), full),
        pl.BlockSpec((1, d), full),
    ]
    args = [efeat, g_stripe, wet, wot, be, bo, lg, lb]
    kwargs = {}
    body = _dense_body
    if prev is not None:
        in_specs.append(pl.BlockSpec(memory_space=pl.ANY))
        args.append(prev)
        kwargs["input_output_aliases"] = {8: 0}
        body = _dense_body_aliased
    return pl.pallas_call(
        body,
        grid=(grid,),
        in_specs=in_specs,
        out_specs=pl.BlockSpec((block_e, d), lambda i: (b0 + i, 0)),
        out_shape=jax.ShapeDtypeStruct((e_total, d), jnp.float32),
        **kwargs,
    )(*args)


def kernel(efeat, nfeat, edge_index, W_e, b_e, W_s, W_d, W_o, b_o, ln_g, ln_b):
    e_total, d = efeat.shape
    n = nfeat.shape[0]
    h = W_e.shape[0]

    src = edge_index[0]
    dst = edge_index[1]

    ps, pd = _project_nodes(nfeat, W_s.T, W_d.T, block_n=1000)

    n_workers = 32
    n_stripes = 5
    stripe = e_total // n_stripes

    gs = []
    for i in range(n_stripes):
        gather = _make_sc_gather(stripe, h, n_workers, chunk=40,
                                 base_off=i * stripe, nslots=10, lead=8)
        gs.append(gather(ps, pd, src, dst))

    wet, wot = W_e.T, W_o.T
    be, bo = b_e.reshape(1, h), b_o.reshape(1, d)
    lg, lb = ln_g.reshape(1, d), ln_b.reshape(1, d)
    out = None
    for i in range(n_stripes):
        out = _dense_mlp_stripe(efeat, gs[i], wet, wot, be, bo, lg, lb,
                                out, i * stripe, block_e=8000)
    return (out, nfeat)
